# baseline (speedup 1.0000x reference)
import jax
import jax.numpy as jnp
import numpy as np
from jax.experimental import pallas as pl
from jax.experimental.pallas import tpu as pltpu

_N, _C, _H, _W = 32, 256, 64, 64
_HW = _H * _W
_EPS = 1e-05

_PERM_NP = np.asarray(
    [31, 7, 4, 29, 16, 19, 2, 5, 30, 3, 22, 6, 18, 10, 11, 15,
     20, 8, 24, 9, 25, 13, 14, 17, 23, 0, 21, 26, 1, 28, 27, 12],
    dtype=np.int32)


def _cycle_plan(perm):
    n = len(perm)
    visited = [False] * n
    load, out_idx, emit_normal, emit_first, save_first = [], [], [], [], []
    prev_cycle_last = None
    for s in range(n):
        if visited[s]:
            continue
        cyc = []
        a = s
        while not visited[a]:
            visited[a] = True
            cyc.append(a)
            a = int(perm[a])
        for m, a in enumerate(cyc):
            load.append(a)
            if m == 0:
                save_first.append(1)
                emit_normal.append(0)
                if prev_cycle_last is None:
                    emit_first.append(0)
                    out_idx.append(-1)
                else:
                    emit_first.append(1)
                    out_idx.append(prev_cycle_last)
            else:
                save_first.append(0)
                emit_first.append(0)
                emit_normal.append(1)
                out_idx.append(cyc[m - 1])
        prev_cycle_last = cyc[-1]
    load.append(load[-1])
    save_first.append(0)
    emit_normal.append(0)
    emit_first.append(1)
    out_idx.append(prev_cycle_last)
    out_idx[0] = out_idx[1]
    idx = np.asarray([load, out_idx], dtype=np.int32)
    flg = np.asarray([emit_normal, emit_first, save_first], dtype=np.int32)
    return idx, flg


_IDX_NP, _FLG_NP = _cycle_plan(_PERM_NP)
_STEPS = _IDX_NP.shape[1]


def _body(idx_ref, flg_ref, x_ref, o_ref,
          xfirst, first_m, first_r, prev_m, prev_s):
    t = pl.program_id(0)
    cur = x_ref[...]
    ones = jnp.ones((1, _C), dtype=jnp.float32)
    s = jax.lax.dot(ones, cur, precision=jax.lax.Precision.HIGHEST)
    sq = jax.lax.dot(ones, cur * cur, precision=jax.lax.Precision.HIGHEST)
    m_cur = s * (1.0 / _C)
    var = (sq - _C * m_cur * m_cur) * (1.0 / (_C - 1))
    s_cur = jnp.sqrt(var + _EPS)
    r_cur = 1.0 / s_cur

    @pl.when(flg_ref[0, t] == 1)
    def _():
        f = r_cur * prev_s[...]
        g = prev_m[...] - m_cur * f
        o_ref[...] = cur * f + g

    @pl.when(flg_ref[1, t] == 1)
    def _():
        f = first_r[...] * prev_s[...]
        g = prev_m[...] - first_m[...] * f
        o_ref[...] = xfirst[...] * f + g

    @pl.when(flg_ref[2, t] == 1)
    def _():
        xfirst[...] = cur
        first_m[...] = m_cur
        first_r[...] = r_cur

    prev_m[...] = m_cur
    prev_s[...] = s_cur


def kernel(x):
    n, c, h, w = x.shape
    xr = x.reshape(n, c, h * w)
    idx = jnp.asarray(_IDX_NP)
    flg = jnp.asarray(_FLG_NP)
    out = pl.pallas_call(
        _body,
        grid_spec=pltpu.PrefetchScalarGridSpec(
            num_scalar_prefetch=2,
            grid=(_STEPS,),
            in_specs=[
                pl.BlockSpec((None, c, _HW), lambda t, i, f: (i[0, t], 0, 0)),
            ],
            out_specs=pl.BlockSpec((None, c, _HW), lambda t, i, f: (i[1, t], 0, 0)),
            scratch_shapes=[
                pltpu.VMEM((c, _HW), jnp.float32),
                pltpu.VMEM((1, _HW), jnp.float32),
                pltpu.VMEM((1, _HW), jnp.float32),
                pltpu.VMEM((1, _HW), jnp.float32),
                pltpu.VMEM((1, _HW), jnp.float32),
            ],
        ),
        out_shape=jax.ShapeDtypeStruct((n, c, h * w), jnp.float32),
    )(idx, flg, xr)

    return out.reshape(n, c, h, w)


# speedup vs baseline: 1.1865x; 1.1865x over previous
import jax
import jax.numpy as jnp
import numpy as np
from jax.experimental import pallas as pl
from jax.experimental.pallas import tpu as pltpu

_N, _C, _H, _W = 32, 256, 64, 64
_HW = _H * _W
_EPS = 1e-05

_PERM_NP = np.asarray(
    [31, 7, 4, 29, 16, 19, 2, 5, 30, 3, 22, 6, 18, 10, 11, 15,
     20, 8, 24, 9, 25, 13, 14, 17, 23, 0, 21, 26, 1, 28, 27, 12],
    dtype=np.int32)


def _cycle_plan(perm):
    n = len(perm)
    visited = [False] * n
    load, out_idx, emit_normal, emit_first, save_first = [], [], [], [], []
    prev_cycle_last = None
    for s in range(n):
        if visited[s]:
            continue
        cyc = []
        a = s
        while not visited[a]:
            visited[a] = True
            cyc.append(a)
            a = int(perm[a])
        for m, a in enumerate(cyc):
            load.append(a)
            if m == 0:
                save_first.append(1)
                emit_normal.append(0)
                if prev_cycle_last is None:
                    emit_first.append(0)
                    out_idx.append(-1)
                else:
                    emit_first.append(1)
                    out_idx.append(prev_cycle_last)
            else:
                save_first.append(0)
                emit_first.append(0)
                emit_normal.append(1)
                out_idx.append(cyc[m - 1])
        prev_cycle_last = cyc[-1]
    load.append(load[-1])
    save_first.append(0)
    emit_normal.append(0)
    emit_first.append(1)
    out_idx.append(prev_cycle_last)
    out_idx[0] = out_idx[1]
    idx = np.asarray([load, out_idx], dtype=np.int32)
    flg = np.asarray([emit_normal, emit_first, save_first], dtype=np.int32)
    return idx, flg


_IDX_NP, _FLG_NP = _cycle_plan(_PERM_NP)
_STEPS = _IDX_NP.shape[1]


def _body(idx_ref, flg_ref, x_ref, o_ref,
          xfirst, first_m, first_r, prev_m, prev_s):
    t = pl.program_id(0)
    cur = x_ref[...]
    s = jnp.sum(cur, axis=0).reshape(1, _HW)
    sq = jnp.sum(cur * cur, axis=0).reshape(1, _HW)
    m_cur = s * (1.0 / _C)
    var = (sq - _C * m_cur * m_cur) * (1.0 / (_C - 1))
    s_cur = jnp.sqrt(var + _EPS)
    r_cur = 1.0 / s_cur

    @pl.when(flg_ref[0, t] == 1)
    def _():
        f = r_cur * prev_s[...]
        g = prev_m[...] - m_cur * f
        o_ref[...] = cur * f + g

    @pl.when(flg_ref[1, t] == 1)
    def _():
        f = first_r[...] * prev_s[...]
        g = prev_m[...] - first_m[...] * f
        o_ref[...] = xfirst[...] * f + g

    @pl.when(flg_ref[2, t] == 1)
    def _():
        xfirst[...] = cur
        first_m[...] = m_cur
        first_r[...] = r_cur

    prev_m[...] = m_cur
    prev_s[...] = s_cur


def kernel(x):
    n, c, h, w = x.shape
    xr = x.reshape(n, c, h * w)
    idx = jnp.asarray(_IDX_NP)
    flg = jnp.asarray(_FLG_NP)
    out = pl.pallas_call(
        _body,
        grid_spec=pltpu.PrefetchScalarGridSpec(
            num_scalar_prefetch=2,
            grid=(_STEPS,),
            in_specs=[
                pl.BlockSpec((None, c, _HW), lambda t, i, f: (i[0, t], 0, 0)),
            ],
            out_specs=pl.BlockSpec((None, c, _HW), lambda t, i, f: (i[1, t], 0, 0)),
            scratch_shapes=[
                pltpu.VMEM((c, _HW), jnp.float32),
                pltpu.VMEM((1, _HW), jnp.float32),
                pltpu.VMEM((1, _HW), jnp.float32),
                pltpu.VMEM((1, _HW), jnp.float32),
                pltpu.VMEM((1, _HW), jnp.float32),
            ],
        ),
        out_shape=jax.ShapeDtypeStruct((n, c, h * w), jnp.float32),
    )(idx, flg, xr)

    return out.reshape(n, c, h, w)
